# baseline (speedup 1.0000x reference)
import functools
import jax
import jax.numpy as jnp
from jax import lax
from jax.experimental import pallas as pl
from jax.experimental.pallas import tpu as pltpu
from jax.experimental.pallas import tpu_sc as plsc

N_NODES = 10000
N_EDGES = 320000
D = 128
G = 128

NC = 2
NS = 16
NW = NC * NS
EPT = N_EDGES // NW
EW_COLS = 80
ROWS_PT = EPT // EW_COLS

_sc_mesh = plsc.VectorSubcoreMesh(core_axis_name="c", subcore_axis_name="s")
_sc_params = pltpu.CompilerParams(needs_layout_passes=False,
                                  use_tc_tiling_on_sc=False)


@functools.partial(
    pl.kernel,
    out_type=jax.ShapeDtypeStruct((NW, N_NODES), jnp.float32),
    mesh=_sc_mesh,
    scratch_types=[
        pltpu.VMEM((ROWS_PT, EW_COLS), jnp.int32),
        pltpu.VMEM((ROWS_PT, EW_COLS), jnp.float32),
        pltpu.VMEM((N_NODES,), jnp.float32),
    ],
    compiler_params=_sc_params,
)
def _sc_deg(dst3d, ew3d, degp, dst_v, ew_v, acc_v):
    cid = lax.axis_index("c")
    sid = lax.axis_index("s")
    wid = cid * NS + sid

    def zbody(i, _):
        acc_v[pl.ds(i * 16, 16)] = jnp.zeros((16,), jnp.float32)
        return 0
    lax.fori_loop(0, N_NODES // 16, zbody, 0)

    pltpu.sync_copy(dst3d.at[wid], dst_v)
    pltpu.sync_copy(ew3d.at[wid], ew_v)

    def ebody(r, _):
        for g in range(EW_COLS // 16):
            idx = dst_v[r, pl.ds(g * 16, 16)]
            val = ew_v[r, pl.ds(g * 16, 16)]
            plsc.addupdate_scatter(acc_v, [idx], val)
        return 0
    lax.fori_loop(0, ROWS_PT, ebody, 0)

    pltpu.sync_copy(acc_v, degp.at[wid])


DH = D // 2
EPT_B = N_EDGES // NS
ROWS_B = EPT_B // EW_COLS
BLK = 80
NBLK = N_NODES // BLK


@functools.partial(
    pl.kernel,
    out_type=jax.ShapeDtypeStruct((NC, N_NODES, DH), jnp.float32),
    mesh=_sc_mesh,
    scratch_types=[
        pltpu.VMEM((ROWS_B, EW_COLS), jnp.int32),
        pltpu.VMEM((ROWS_B, EW_COLS), jnp.int32),
        pltpu.VMEM((ROWS_B, EW_COLS), jnp.float32),
        [pltpu.VMEM((EW_COLS, DH), jnp.float32) for _ in range(5)],
        pltpu.VMEM((BLK, DH), jnp.float32),
        pltpu.VMEM_SHARED((N_NODES, DH), jnp.float32),
        [pltpu.SemaphoreType.DMA for _ in range(5)],
        [pltpu.SemaphoreType.DMA for _ in range(5)],
    ],
    compiler_params=_sc_params,
)
def _sc_agg(z2_hbm, src3d, dst3d, ew3d, aggp,
            src_v, dst_v, ew_v, rows, zblk_v, acc_sh, gsem, ssem):
    cid = lax.axis_index("c")
    sid = lax.axis_index("s")
    zc = z2_hbm.at[cid]

    pltpu.sync_copy(src3d.at[sid], src_v)
    pltpu.sync_copy(dst3d.at[sid], dst_v)
    pltpu.sync_copy(ew3d.at[sid], ew_v)

    pltpu.async_copy(zc.at[src_v.at[0]], rows[0], gsem[0])
    pltpu.async_copy(zc.at[src_v.at[1]], rows[1], gsem[1])

    def zb(r, _):
        for g in range(DH // 16):
            zblk_v[r, pl.ds(g * 16, 16)] = jnp.zeros((16,), jnp.float32)
        return 0
    lax.fori_loop(0, BLK, zb, 0)

    def zacc(k, _):
        b = sid + k * NS
        @pl.when(b < NBLK)
        def _():
            pltpu.sync_copy(zblk_v, acc_sh.at[pl.ds(b * BLK, BLK)])
        return 0
    lax.fori_loop(0, (NBLK + NS - 1) // NS, zacc, 0)
    plsc.subcore_barrier()

    def scale(rows_b, c):
        cvec = jnp.full((16,), c, jnp.int32)

        @plsc.parallel_loop(0, EW_COLS, unroll=8)
        def _(j):
            ewb = plsc.load_gather(ew_v, [cvec, jnp.full((16,), j, jnp.int32)])
            for g in range(DH // 16):
                sl = pl.ds(g * 16, 16)
                rows_b[j, sl] = rows_b[j, sl] * ewb

    def ring(k, _):
        for b in range(5):
            c = 5 * k + b
            pltpu.make_async_copy(zc.at[src_v.at[c]], rows[b], gsem[b]).wait()
            scale(rows[b], c)
            bg = (b + 2) % 5
            @pl.when(c + 2 < ROWS_B)
            def _():
                pltpu.async_copy(zc.at[src_v.at[c + 2]], rows[bg], gsem[bg])
        return 0
    lax.fori_loop(0, ROWS_B // 5, ring, 0)

    plsc.subcore_barrier()

    def wout(k, _):
        b = sid + k * NS
        @pl.when(b < NBLK)
        def _():
            pltpu.sync_copy(acc_sh.at[pl.ds(b * BLK, BLK)],
                            aggp.at[cid, pl.ds(b * BLK, BLK)])
        return 0
    lax.fori_loop(0, (NBLK + NS - 1) // NS, wout, 0)


def _k1_body(degp_ref, x_ref, z_ref, dinv_ref):
    deg = jnp.sum(degp_ref[...], axis=0)[:, None] + 1.0
    dinv = jnp.where(deg > 0, lax.rsqrt(deg), 0.0)
    dinv_ref[...] = dinv
    z = dinv * x_ref[...]
    z_ref[0] = z[:, :D // 2]
    z_ref[1] = z[:, D // 2:]


def _run_k1(deg_partials, x):
    return pl.pallas_call(
        _k1_body,
        out_shape=(
            jax.ShapeDtypeStruct((NC, N_NODES, D // 2), jnp.float32),
            jax.ShapeDtypeStruct((N_NODES, 1), jnp.float32),
        ),
    )(deg_partials, x)


def _k2_body(agg_ref, x_ref, dinv_ref, We_ref, be_ref, Wp_ref, bp_ref,
             counts_ref, out_ref):
    dinv = dinv_ref[...]
    agg = jnp.concatenate([agg_ref[0], agg_ref[1]], axis=1)
    h = dinv * agg + (dinv * dinv) * x_ref[...]
    embed = jax.nn.relu(
        jnp.dot(h, We_ref[...], preferred_element_type=jnp.float32) + be_ref[...])
    score = jnp.dot(h, Wp_ref[...], preferred_element_type=jnp.float32) \
        + bp_ref[...]

    m = jnp.max(score)
    e = jnp.exp(score - m)

    counts = counts_ref[...]
    jj = lax.broadcasted_iota(jnp.int32, (G, G), 0)
    gg = lax.broadcasted_iota(jnp.int32, (G, G), 1)
    ut = (jj <= gg).astype(jnp.float32)
    ends = jnp.dot(counts, ut, preferred_element_type=jnp.float32)
    starts = ends - counts
    gidx = lax.broadcasted_iota(jnp.int32, (1, G), 1)
    valid = (gidx < 100).astype(jnp.float32)

    v = lax.broadcasted_iota(jnp.int32, (N_NODES, G), 0).astype(jnp.float32)
    M = ((v >= starts) & (v < ends)).astype(jnp.float32) * valid
    Me = M * e

    dn = (((0,), (0,)), ((), ()))
    num = lax.dot_general(Me, embed, dn, preferred_element_type=jnp.float32)
    ones = jnp.ones((N_NODES, 1), jnp.float32)
    zsum = lax.dot_general(Me, ones, dn, preferred_element_type=jnp.float32)
    out_ref[...] = num / jnp.where(zsum > 0, zsum, 1.0)


def _run_k2(agg, x, dinv, W_embed, b_embed, W_pool, b_pool, counts_row):
    return pl.pallas_call(
        _k2_body,
        out_shape=jax.ShapeDtypeStruct((G, D), jnp.float32),
    )(agg, x, dinv, W_embed, b_embed, W_pool, b_pool, counts_row)


def kernel(x, edge_index, edge_weight, topo, W_embed, b_embed, W_pool, b_pool):
    src = edge_index[0]
    dst = edge_index[1]
    dst3d = dst.reshape(NW, ROWS_PT, EW_COLS)
    ew3d = edge_weight.reshape(NW, ROWS_PT, EW_COLS)

    degp = _sc_deg(dst3d, ew3d)

    z, dinv = _run_k1(degp, x)

    srcB = src.reshape(NS, ROWS_B, EW_COLS)
    dstB = dst.reshape(NS, ROWS_B, EW_COLS)
    ewB = edge_weight.reshape(NS, ROWS_B, EW_COLS)
    agg = _sc_agg(z, srcB, dstB, ewB)

    counts_row = jnp.zeros((1, G), jnp.float32).at[0, :100].set(
        topo[:, 0].astype(jnp.float32))
    be = b_embed[None, :]
    bp = b_pool[None, :]
    pooled = _run_k2(agg, x, dinv, W_embed, be, W_pool, bp, counts_row)
    return pooled[:100]


# speedup vs baseline: 1.1593x; 1.1593x over previous
import functools
import jax
import jax.numpy as jnp
from jax import lax
from jax.experimental import pallas as pl
from jax.experimental.pallas import tpu as pltpu
from jax.experimental.pallas import tpu_sc as plsc

N_NODES = 10000
N_EDGES = 320000
D = 128
G = 128

NC = 2
NS = 16
NW = NC * NS
EPT = N_EDGES // NW
EW_COLS = 80
ROWS_PT = EPT // EW_COLS

_sc_mesh = plsc.VectorSubcoreMesh(core_axis_name="c", subcore_axis_name="s")
_sc_params = pltpu.CompilerParams(needs_layout_passes=False,
                                  use_tc_tiling_on_sc=False)


@functools.partial(
    pl.kernel,
    out_type=jax.ShapeDtypeStruct((NW, N_NODES), jnp.float32),
    mesh=_sc_mesh,
    scratch_types=[
        pltpu.VMEM((ROWS_PT, EW_COLS), jnp.int32),
        pltpu.VMEM((ROWS_PT, EW_COLS), jnp.float32),
        pltpu.VMEM((N_NODES,), jnp.float32),
    ],
    compiler_params=_sc_params,
)
def _sc_deg(dst3d, ew3d, degp, dst_v, ew_v, acc_v):
    cid = lax.axis_index("c")
    sid = lax.axis_index("s")
    wid = cid * NS + sid

    def zbody(i, _):
        acc_v[pl.ds(i * 16, 16)] = jnp.zeros((16,), jnp.float32)
        return 0
    lax.fori_loop(0, N_NODES // 16, zbody, 0)

    pltpu.sync_copy(dst3d.at[wid], dst_v)
    pltpu.sync_copy(ew3d.at[wid], ew_v)

    def ebody(r, _):
        for g in range(EW_COLS // 16):
            idx = dst_v[r, pl.ds(g * 16, 16)]
            val = ew_v[r, pl.ds(g * 16, 16)]
            plsc.addupdate_scatter(acc_v, [idx], val)
        return 0
    lax.fori_loop(0, ROWS_PT, ebody, 0)

    pltpu.sync_copy(acc_v, degp.at[wid])


DH = D // 2
EPT_B = N_EDGES // NS
ROWS_B = EPT_B // EW_COLS
BLK = 80
NBLK = N_NODES // BLK


@functools.partial(
    pl.kernel,
    out_type=jax.ShapeDtypeStruct((NC, N_NODES, DH), jnp.float32),
    mesh=_sc_mesh,
    scratch_types=[
        pltpu.VMEM((ROWS_B, EW_COLS), jnp.int32),
        pltpu.VMEM((ROWS_B, EW_COLS), jnp.int32),
        pltpu.VMEM((ROWS_B, EW_COLS), jnp.float32),
        [pltpu.VMEM((EW_COLS, DH), jnp.float32) for _ in range(5)],
        pltpu.VMEM((BLK, DH), jnp.float32),
        pltpu.VMEM_SHARED((N_NODES, DH), jnp.float32),
        [pltpu.SemaphoreType.DMA for _ in range(5)],
        [pltpu.SemaphoreType.DMA for _ in range(5)],
    ],
    compiler_params=_sc_params,
)
def _sc_agg(z2_hbm, src3d, dst3d, ew3d, aggp,
            src_v, dst_v, ew_v, rows, zblk_v, acc_sh, gsem, ssem):
    cid = lax.axis_index("c")
    sid = lax.axis_index("s")
    zc = z2_hbm.at[cid]

    pltpu.sync_copy(src3d.at[sid], src_v)
    pltpu.sync_copy(dst3d.at[sid], dst_v)
    pltpu.sync_copy(ew3d.at[sid], ew_v)

    pltpu.async_copy(zc.at[src_v.at[0]], rows[0], gsem[0])
    pltpu.async_copy(zc.at[src_v.at[1]], rows[1], gsem[1])
    pltpu.async_copy(zc.at[src_v.at[2]], rows[2], gsem[2])

    def zb(r, _):
        for g in range(DH // 16):
            zblk_v[r, pl.ds(g * 16, 16)] = jnp.zeros((16,), jnp.float32)
        return 0
    lax.fori_loop(0, BLK, zb, 0)

    def zacc(k, _):
        b = sid + k * NS
        @pl.when(b < NBLK)
        def _():
            pltpu.sync_copy(zblk_v, acc_sh.at[pl.ds(b * BLK, BLK)])
        return 0
    lax.fori_loop(0, (NBLK + NS - 1) // NS, zacc, 0)
    plsc.subcore_barrier()

    def scale(rows_b, c):
        cvec = jnp.full((16,), c, jnp.int32)

        @plsc.parallel_loop(0, EW_COLS, unroll=8)
        def _(j):
            ewb = plsc.load_gather(ew_v, [cvec, jnp.full((16,), j, jnp.int32)])
            for g in range(DH // 16):
                sl = pl.ds(g * 16, 16)
                rows_b[j, sl] = rows_b[j, sl] * ewb

    def ring(k, _):
        for b in range(5):
            c = 5 * k + b
            pltpu.make_async_copy(zc.at[src_v.at[c]], rows[b], gsem[b]).wait()
            scale(rows[b], c)
            pltpu.async_copy(rows[b], acc_sh.at[dst_v.at[c]], ssem[b],
                             add=True)
            bg = (b + 3) % 5
            @pl.when(c >= 2)
            def _():
                pltpu.make_async_copy(
                    rows[bg], acc_sh.at[dst_v.at[jnp.maximum(c - 2, 0)]],
                    ssem[bg]).wait()
            @pl.when(c + 3 < ROWS_B)
            def _():
                pltpu.async_copy(zc.at[src_v.at[c + 3]], rows[bg], gsem[bg])
        return 0
    lax.fori_loop(0, ROWS_B // 5, ring, 0)

    for b in (3, 4):
        pltpu.make_async_copy(rows[b], acc_sh.at[dst_v.at[ROWS_B - 5 + b]],
                              ssem[b]).wait()

    plsc.subcore_barrier()

    def wout(k, _):
        b = sid + k * NS
        @pl.when(b < NBLK)
        def _():
            pltpu.sync_copy(acc_sh.at[pl.ds(b * BLK, BLK)],
                            aggp.at[cid, pl.ds(b * BLK, BLK)])
        return 0
    lax.fori_loop(0, (NBLK + NS - 1) // NS, wout, 0)


def _k1_body(degp_ref, x_ref, z_ref, dinv_ref):
    deg = jnp.sum(degp_ref[...], axis=0)[:, None] + 1.0
    dinv = jnp.where(deg > 0, lax.rsqrt(deg), 0.0)
    dinv_ref[...] = dinv
    z = dinv * x_ref[...]
    z_ref[0] = z[:, :D // 2]
    z_ref[1] = z[:, D // 2:]


def _run_k1(deg_partials, x):
    return pl.pallas_call(
        _k1_body,
        out_shape=(
            jax.ShapeDtypeStruct((NC, N_NODES, D // 2), jnp.float32),
            jax.ShapeDtypeStruct((N_NODES, 1), jnp.float32),
        ),
    )(deg_partials, x)


def _k2_body(agg_ref, x_ref, dinv_ref, We_ref, be_ref, Wp_ref, bp_ref,
             counts_ref, out_ref):
    dinv = dinv_ref[...]
    agg = jnp.concatenate([agg_ref[0], agg_ref[1]], axis=1)
    h = dinv * agg + (dinv * dinv) * x_ref[...]
    embed = jax.nn.relu(
        jnp.dot(h, We_ref[...], preferred_element_type=jnp.float32) + be_ref[...])
    score = jnp.dot(h, Wp_ref[...], preferred_element_type=jnp.float32) \
        + bp_ref[...]

    m = jnp.max(score)
    e = jnp.exp(score - m)

    counts = counts_ref[...]
    jj = lax.broadcasted_iota(jnp.int32, (G, G), 0)
    gg = lax.broadcasted_iota(jnp.int32, (G, G), 1)
    ut = (jj <= gg).astype(jnp.float32)
    ends = jnp.dot(counts, ut, preferred_element_type=jnp.float32)
    starts = ends - counts
    gidx = lax.broadcasted_iota(jnp.int32, (1, G), 1)
    valid = (gidx < 100).astype(jnp.float32)

    v = lax.broadcasted_iota(jnp.int32, (N_NODES, G), 0).astype(jnp.float32)
    M = ((v >= starts) & (v < ends)).astype(jnp.float32) * valid
    Me = M * e

    dn = (((0,), (0,)), ((), ()))
    num = lax.dot_general(Me, embed, dn, preferred_element_type=jnp.float32)
    ones = jnp.ones((N_NODES, 1), jnp.float32)
    zsum = lax.dot_general(Me, ones, dn, preferred_element_type=jnp.float32)
    out_ref[...] = num / jnp.where(zsum > 0, zsum, 1.0)


def _run_k2(agg, x, dinv, W_embed, b_embed, W_pool, b_pool, counts_row):
    return pl.pallas_call(
        _k2_body,
        out_shape=jax.ShapeDtypeStruct((G, D), jnp.float32),
    )(agg, x, dinv, W_embed, b_embed, W_pool, b_pool, counts_row)


def kernel(x, edge_index, edge_weight, topo, W_embed, b_embed, W_pool, b_pool):
    src = edge_index[0]
    dst = edge_index[1]
    dst3d = dst.reshape(NW, ROWS_PT, EW_COLS)
    ew3d = edge_weight.reshape(NW, ROWS_PT, EW_COLS)

    degp = _sc_deg(dst3d, ew3d)

    z, dinv = _run_k1(degp, x)

    srcB = src.reshape(NS, ROWS_B, EW_COLS)
    dstB = dst.reshape(NS, ROWS_B, EW_COLS)
    ewB = edge_weight.reshape(NS, ROWS_B, EW_COLS)
    agg = _sc_agg(z, srcB, dstB, ewB)

    counts_row = jnp.zeros((1, G), jnp.float32).at[0, :100].set(
        topo[:, 0].astype(jnp.float32))
    be = b_embed[None, :]
    bp = b_pool[None, :]
    pooled = _run_k2(agg, x, dinv, W_embed, be, W_pool, bp, counts_row)
    return pooled[:100]


# speedup vs baseline: 1.1693x; 1.0087x over previous
import functools
import jax
import jax.numpy as jnp
from jax import lax
from jax.experimental import pallas as pl
from jax.experimental.pallas import tpu as pltpu
from jax.experimental.pallas import tpu_sc as plsc

N_NODES = 10000
N_EDGES = 320000
D = 128
G = 128

NC = 2
NS = 16
NW = NC * NS
EPT = N_EDGES // NW
EW_COLS = 80
ROWS_PT = EPT // EW_COLS

_sc_mesh = plsc.VectorSubcoreMesh(core_axis_name="c", subcore_axis_name="s")
_sc_params = pltpu.CompilerParams(needs_layout_passes=False,
                                  use_tc_tiling_on_sc=False)


@functools.partial(
    pl.kernel,
    out_type=jax.ShapeDtypeStruct((NW, N_NODES), jnp.float32),
    mesh=_sc_mesh,
    scratch_types=[
        pltpu.VMEM((ROWS_PT, EW_COLS), jnp.int32),
        pltpu.VMEM((ROWS_PT, EW_COLS), jnp.float32),
        pltpu.VMEM((N_NODES,), jnp.float32),
    ],
    compiler_params=_sc_params,
)
def _sc_deg(dst3d, ew3d, degp, dst_v, ew_v, acc_v):
    cid = lax.axis_index("c")
    sid = lax.axis_index("s")
    wid = cid * NS + sid

    def zbody(i, _):
        acc_v[pl.ds(i * 16, 16)] = jnp.zeros((16,), jnp.float32)
        return 0
    lax.fori_loop(0, N_NODES // 16, zbody, 0)

    pltpu.sync_copy(dst3d.at[wid], dst_v)
    pltpu.sync_copy(ew3d.at[wid], ew_v)

    def ebody(r, _):
        for g in range(EW_COLS // 16):
            idx = dst_v[r, pl.ds(g * 16, 16)]
            val = ew_v[r, pl.ds(g * 16, 16)]
            plsc.addupdate_scatter(acc_v, [idx], val)
        return 0
    lax.fori_loop(0, ROWS_PT, ebody, 0)

    pltpu.sync_copy(acc_v, degp.at[wid])


DH = D // 2
EPT_B = N_EDGES // NS
ROWS_B = EPT_B // EW_COLS
BLK = 80
NBLK = N_NODES // BLK


@functools.partial(
    pl.kernel,
    out_type=jax.ShapeDtypeStruct((NC, N_NODES, DH), jnp.float32),
    mesh=_sc_mesh,
    scratch_types=[
        pltpu.VMEM((ROWS_B, EW_COLS), jnp.int32),
        pltpu.VMEM((ROWS_B, EW_COLS), jnp.int32),
        pltpu.VMEM((ROWS_B, EW_COLS), jnp.float32),
        [pltpu.VMEM((EW_COLS, DH), jnp.float32) for _ in range(5)],
        pltpu.VMEM((BLK, DH), jnp.float32),
        pltpu.VMEM_SHARED((N_NODES, DH), jnp.float32),
        [pltpu.SemaphoreType.DMA for _ in range(5)],
        [pltpu.SemaphoreType.DMA for _ in range(5)],
    ],
    compiler_params=_sc_params,
)
def _sc_agg(z2_hbm, src3d, dst3d, ew3d, aggp,
            src_v, dst_v, ew_v, rows, zblk_v, acc_sh, gsem, ssem):
    cid = lax.axis_index("c")
    sid = lax.axis_index("s")
    zc = z2_hbm.at[cid]

    pltpu.sync_copy(src3d.at[sid], src_v)
    pltpu.sync_copy(dst3d.at[sid], dst_v)
    pltpu.sync_copy(ew3d.at[sid], ew_v)

    pltpu.async_copy(zc.at[src_v.at[0]], rows[0], gsem[0])
    pltpu.async_copy(zc.at[src_v.at[1]], rows[1], gsem[1])
    pltpu.async_copy(zc.at[src_v.at[2]], rows[2], gsem[2])
    pltpu.async_copy(zc.at[src_v.at[3]], rows[3], gsem[3])

    def zb(r, _):
        for g in range(DH // 16):
            zblk_v[r, pl.ds(g * 16, 16)] = jnp.zeros((16,), jnp.float32)
        return 0
    lax.fori_loop(0, BLK, zb, 0)

    def zacc(k, _):
        b = sid + k * NS
        @pl.when(b < NBLK)
        def _():
            pltpu.sync_copy(zblk_v, acc_sh.at[pl.ds(b * BLK, BLK)])
        return 0
    lax.fori_loop(0, (NBLK + NS - 1) // NS, zacc, 0)
    plsc.subcore_barrier()

    def scale(rows_b, c):
        cvec = jnp.full((16,), c, jnp.int32)

        @plsc.parallel_loop(0, EW_COLS, unroll=8)
        def _(j):
            ewb = plsc.load_gather(ew_v, [cvec, jnp.full((16,), j, jnp.int32)])
            for g in range(DH // 16):
                sl = pl.ds(g * 16, 16)
                rows_b[j, sl] = rows_b[j, sl] * ewb

    def ring(k, _):
        for b in range(5):
            c = 5 * k + b
            pltpu.make_async_copy(zc.at[src_v.at[c]], rows[b], gsem[b]).wait()
            scale(rows[b], c)
            pltpu.async_copy(rows[b], acc_sh.at[dst_v.at[c]], ssem[b],
                             add=True)
            bg = (b + 4) % 5
            @pl.when(c >= 1)
            def _():
                pltpu.make_async_copy(
                    rows[bg], acc_sh.at[dst_v.at[jnp.maximum(c - 1, 0)]],
                    ssem[bg]).wait()
            @pl.when(c + 4 < ROWS_B)
            def _():
                pltpu.async_copy(zc.at[src_v.at[c + 4]], rows[bg], gsem[bg])
        return 0
    lax.fori_loop(0, ROWS_B // 5, ring, 0)

    for b in (4,):
        pltpu.make_async_copy(rows[b], acc_sh.at[dst_v.at[ROWS_B - 5 + b]],
                              ssem[b]).wait()

    plsc.subcore_barrier()

    def wout(k, _):
        b = sid + k * NS
        @pl.when(b < NBLK)
        def _():
            pltpu.sync_copy(acc_sh.at[pl.ds(b * BLK, BLK)],
                            aggp.at[cid, pl.ds(b * BLK, BLK)])
        return 0
    lax.fori_loop(0, (NBLK + NS - 1) // NS, wout, 0)


def _k1_body(degp_ref, x_ref, z_ref, dinv_ref):
    deg = jnp.sum(degp_ref[...], axis=0)[:, None] + 1.0
    dinv = jnp.where(deg > 0, lax.rsqrt(deg), 0.0)
    dinv_ref[...] = dinv
    z = dinv * x_ref[...]
    z_ref[0] = z[:, :D // 2]
    z_ref[1] = z[:, D // 2:]


def _run_k1(deg_partials, x):
    return pl.pallas_call(
        _k1_body,
        out_shape=(
            jax.ShapeDtypeStruct((NC, N_NODES, D // 2), jnp.float32),
            jax.ShapeDtypeStruct((N_NODES, 1), jnp.float32),
        ),
    )(deg_partials, x)


def _k2_body(agg_ref, x_ref, dinv_ref, We_ref, be_ref, Wp_ref, bp_ref,
             counts_ref, out_ref):
    dinv = dinv_ref[...]
    agg = jnp.concatenate([agg_ref[0], agg_ref[1]], axis=1)
    h = dinv * agg + (dinv * dinv) * x_ref[...]
    embed = jax.nn.relu(
        jnp.dot(h, We_ref[...], preferred_element_type=jnp.float32) + be_ref[...])
    score = jnp.dot(h, Wp_ref[...], preferred_element_type=jnp.float32) \
        + bp_ref[...]

    m = jnp.max(score)
    e = jnp.exp(score - m)

    counts = counts_ref[...]
    jj = lax.broadcasted_iota(jnp.int32, (G, G), 0)
    gg = lax.broadcasted_iota(jnp.int32, (G, G), 1)
    ut = (jj <= gg).astype(jnp.float32)
    ends = jnp.dot(counts, ut, preferred_element_type=jnp.float32)
    starts = ends - counts
    gidx = lax.broadcasted_iota(jnp.int32, (1, G), 1)
    valid = (gidx < 100).astype(jnp.float32)

    v = lax.broadcasted_iota(jnp.int32, (N_NODES, G), 0).astype(jnp.float32)
    M = ((v >= starts) & (v < ends)).astype(jnp.float32) * valid
    Me = M * e

    dn = (((0,), (0,)), ((), ()))
    num = lax.dot_general(Me, embed, dn, preferred_element_type=jnp.float32)
    ones = jnp.ones((N_NODES, 1), jnp.float32)
    zsum = lax.dot_general(Me, ones, dn, preferred_element_type=jnp.float32)
    out_ref[...] = num / jnp.where(zsum > 0, zsum, 1.0)


def _run_k2(agg, x, dinv, W_embed, b_embed, W_pool, b_pool, counts_row):
    return pl.pallas_call(
        _k2_body,
        out_shape=jax.ShapeDtypeStruct((G, D), jnp.float32),
    )(agg, x, dinv, W_embed, b_embed, W_pool, b_pool, counts_row)


def kernel(x, edge_index, edge_weight, topo, W_embed, b_embed, W_pool, b_pool):
    src = edge_index[0]
    dst = edge_index[1]
    dst3d = dst.reshape(NW, ROWS_PT, EW_COLS)
    ew3d = edge_weight.reshape(NW, ROWS_PT, EW_COLS)

    degp = _sc_deg(dst3d, ew3d)

    z, dinv = _run_k1(degp, x)

    srcB = src.reshape(NS, ROWS_B, EW_COLS)
    dstB = dst.reshape(NS, ROWS_B, EW_COLS)
    ewB = edge_weight.reshape(NS, ROWS_B, EW_COLS)
    agg = _sc_agg(z, srcB, dstB, ewB)

    counts_row = jnp.zeros((1, G), jnp.float32).at[0, :100].set(
        topo[:, 0].astype(jnp.float32))
    be = b_embed[None, :]
    bp = b_pool[None, :]
    pooled = _run_k2(agg, x, dinv, W_embed, be, W_pool, bp, counts_row)
    return pooled[:100]
